# baseline (speedup 1.0000x reference)
import functools

import jax
import jax.numpy as jnp
from jax import lax
from jax.experimental import pallas as pl
from jax.experimental.pallas import tpu as pltpu
from jax.experimental.pallas import tpu_sc as plsc

N = 10000
E = 320000
D = 128
L = 128
R = E // L
NC, NS = 2, 16
NW = NC * NS
RPW = R // NW
REM = R - RPW * NW

GPW = 1250 // NS
GREM = 1250 - GPW * NS
_CHUNKS = ((0, 128), (128, 128), (256, 128), (384, 128), (512, 112))

_MESH = dict(core_axis_name="c", subcore_axis_name="s", num_cores=NC,
             num_subcores=NS)


def _worker_id():
    return lax.axis_index("s") * NC + lax.axis_index("c")


@functools.partial(
    pl.kernel,
    out_type=jax.ShapeDtypeStruct((NC, N, D), jnp.float32),
    mesh=plsc.VectorSubcoreMesh(**_MESH),
    scratch_types=[
        pltpu.VMEM((RPW, 1, L), jnp.int32),
        pltpu.VMEM((2, L, D), jnp.float32),
        pltpu.VMEM_SHARED((N, D), jnp.float32),
        pltpu.SemaphoreType.DMA,
        pltpu.SemaphoreType.DMA,
    ],
)
def _scatter(efeat_hbm, dst_hbm, mpart_hbm, idx_v, rows_v, m_sh, sem0, sem1):
    c = lax.axis_index("c")
    s = lax.axis_index("s")
    wid = _worker_id()
    sems = (sem0, sem1)
    zero = jnp.zeros((16,), jnp.float32)

    def zr(i, carry):
        for k in range(D // 16):
            rows_v[0, i, pl.ds(k * 16, 16)] = zero
        return carry

    lax.fori_loop(0, L, zr, 0)
    base_m = s * GPW * 8
    for off, n in _CHUNKS:
        pltpu.sync_copy(rows_v.at[0, pl.ds(0, n)],
                        m_sh.at[pl.ds(base_m + off, n)])

    @pl.when(s < GREM)
    def _():
        pltpu.sync_copy(rows_v.at[0, pl.ds(0, 8)],
                        m_sh.at[pl.ds((NS * GPW + s) * 8, 8)])

    plsc.subcore_barrier()

    base = wid * RPW
    pltpu.sync_copy(dst_hbm.at[pl.ds(base, RPW)], idx_v)
    pltpu.async_copy(efeat_hbm.at[pl.ds(base * L, L)], rows_v.at[0], sem0)

    def pair(g, carry):
        for b in (0, 1):
            j = 2 * g + b
            pltpu.make_async_copy(efeat_hbm.at[pl.ds((base + j) * L, L)],
                                  rows_v.at[b], sems[b]).wait()

            @pl.when(j + 1 < RPW)
            def _():
                pltpu.async_copy(
                    efeat_hbm.at[pl.ds((base + j + 1) * L, L)],
                    rows_v.at[1 - b], sems[1 - b])

            pltpu.sync_copy(rows_v.at[b], m_sh.at[idx_v.at[j, 0]], add=True)
        return carry

    lax.fori_loop(0, RPW // 2, pair, 0)

    @pl.when(wid < REM)
    def _():
        r = NW * RPW + wid
        pltpu.sync_copy(dst_hbm.at[pl.ds(r, 1)], idx_v.at[pl.ds(0, 1)])
        pltpu.sync_copy(efeat_hbm.at[pl.ds(r * L, L)], rows_v.at[0])
        pltpu.sync_copy(rows_v.at[0], m_sh.at[idx_v.at[0, 0]], add=True)

    plsc.subcore_barrier()

    for off, n in _CHUNKS:
        pltpu.sync_copy(m_sh.at[pl.ds(base_m + off, n)],
                        rows_v.at[0, pl.ds(0, n)])
        pltpu.sync_copy(rows_v.at[0, pl.ds(0, n)],
                        mpart_hbm.at[c, pl.ds(base_m + off, n)])

    @pl.when(s < GREM)
    def _():
        row = (NS * GPW + s) * 8
        pltpu.sync_copy(m_sh.at[pl.ds(row, 8)], rows_v.at[0, pl.ds(0, 8)])
        pltpu.sync_copy(rows_v.at[0, pl.ds(0, 8)],
                        mpart_hbm.at[c, pl.ds(row, 8)])


def _mw_body(mp_ref, w_ref, mw_ref):
    m = mp_ref[0] + mp_ref[1]
    mw_ref[...] = lax.dot_general(
        m, w_ref[...], (((1,), (1,)), ((), ())),
        preferred_element_type=jnp.float32)


_mw = pl.pallas_call(
    _mw_body,
    out_shape=jax.ShapeDtypeStruct((N, D), jnp.float32),
)


@functools.partial(
    pl.kernel,
    out_type=jax.ShapeDtypeStruct((E, D), jnp.float32),
    mesh=plsc.VectorSubcoreMesh(**_MESH),
    scratch_types=[
        pltpu.VMEM((RPW, 1, L), jnp.int32),
        pltpu.VMEM((2, L, D), jnp.float32),
        pltpu.VMEM_SHARED((N, D), jnp.float32),
        pltpu.SemaphoreType.DMA,
        pltpu.SemaphoreType.DMA,
    ],
)
def _gather(m_hbm, src_hbm, g_hbm, idx_v, rows_v, m_sh, sem0, sem1):
    s = lax.axis_index("s")
    wid = _worker_id()
    sems = (sem0, sem1)
    base_m = s * GPW * 8
    for off, n in _CHUNKS:
        pltpu.sync_copy(m_hbm.at[pl.ds(base_m + off, n)],
                        rows_v.at[0, pl.ds(0, n)])
        pltpu.sync_copy(rows_v.at[0, pl.ds(0, n)],
                        m_sh.at[pl.ds(base_m + off, n)])

    @pl.when(s < GREM)
    def _():
        row = (NS * GPW + s) * 8
        pltpu.sync_copy(m_hbm.at[pl.ds(row, 8)], rows_v.at[0, pl.ds(0, 8)])
        pltpu.sync_copy(rows_v.at[0, pl.ds(0, 8)], m_sh.at[pl.ds(row, 8)])

    plsc.subcore_barrier()
    base = wid * RPW
    pltpu.sync_copy(src_hbm.at[pl.ds(base, RPW)], idx_v)
    pltpu.async_copy(m_sh.at[idx_v.at[0, 0]], rows_v.at[0], sem0)

    def pair(g, carry):
        for b in (0, 1):
            j = 2 * g + b
            pltpu.make_async_copy(m_sh.at[idx_v.at[j, 0]],
                                  rows_v.at[b], sems[b]).wait()

            @pl.when(j + 1 < RPW)
            def _():
                pltpu.async_copy(m_sh.at[idx_v.at[j + 1, 0]],
                                 rows_v.at[1 - b], sems[1 - b])

            pltpu.sync_copy(rows_v.at[b], g_hbm.at[pl.ds((base + j) * L, L)])
        return carry

    lax.fori_loop(0, RPW // 2, pair, 0)

    @pl.when(wid < REM)
    def _():
        r = NW * RPW + wid
        pltpu.sync_copy(src_hbm.at[pl.ds(r, 1)], idx_v.at[pl.ds(0, 1)])
        pltpu.async_copy(m_sh.at[idx_v.at[0, 0]], rows_v.at[0], sem0).wait()
        pltpu.sync_copy(rows_v.at[0], g_hbm.at[pl.ds(r * L, L)])


BE = 3200


def _tkern_body(init_ref, ef_ref, w_ref, b_ref, t_ref):
    t_ref[...] = (init_ref[...] + b_ref[...] - lax.dot_general(
        ef_ref[...], w_ref[...], (((1,), (1,)), ((), ())),
        preferred_element_type=jnp.float32)).astype(jnp.bfloat16)


_tkern = pl.pallas_call(
    _tkern_body,
    grid=(E // BE,),
    in_specs=[
        pl.BlockSpec((BE, D), lambda i: (i, 0)),
        pl.BlockSpec((BE, D), lambda i: (i, 0)),
        pl.BlockSpec((D, D), lambda i: (0, 0)),
        pl.BlockSpec((1, D), lambda i: (0, 0)),
    ],
    out_specs=pl.BlockSpec((BE, D), lambda i: (i, 0)),
    out_shape=jax.ShapeDtypeStruct((E, D), jnp.bfloat16),
)


def _add_body(t_ref, g_ref, out_ref):
    out_ref[...] = t_ref[...].astype(jnp.float32) + g_ref[...]


_final_add = pl.pallas_call(
    _add_body,
    grid=(E // BE,),
    in_specs=[
        pl.BlockSpec((BE, D), lambda i: (i, 0)),
        pl.BlockSpec((BE, D), lambda i: (i, 0)),
    ],
    out_specs=pl.BlockSpec((BE, D), lambda i: (i, 0)),
    out_shape=jax.ShapeDtypeStruct((E, D), jnp.float32),
)


def kernel(efeat, initial_efeat, W, b, edge_index):
    ei = edge_index.astype(jnp.int32)
    src = ei[0].reshape(R, 1, L)
    dst = ei[1].reshape(R, 1, L)
    mpart = _scatter(efeat, dst)
    t = _tkern(initial_efeat, efeat, W, b.reshape(1, D))
    mw = _mw(mpart, W)
    g = _gather(mw, src)
    return _final_add(t, g)


# speedup vs baseline: 1.1163x; 1.1163x over previous
import functools

import jax
import jax.numpy as jnp
from jax import lax
from jax.experimental import pallas as pl
from jax.experimental.pallas import tpu as pltpu
from jax.experimental.pallas import tpu_sc as plsc

N = 10000
E = 320000
D = 128
L = 128
R = E // L
NC, NS = 2, 16
NW = NC * NS
RPW = R // NW
REM = R - RPW * NW

GPW = 1250 // NS
GREM = 1250 - GPW * NS
_CHUNKS = ((0, 128), (128, 128), (256, 128), (384, 128), (512, 112))

_MESH = dict(core_axis_name="c", subcore_axis_name="s", num_cores=NC,
             num_subcores=NS)


def _worker_id():
    return lax.axis_index("s") * NC + lax.axis_index("c")


@functools.partial(
    pl.kernel,
    out_type=jax.ShapeDtypeStruct((NC, N, D), jnp.float32),
    mesh=plsc.VectorSubcoreMesh(**_MESH),
    scratch_types=[
        pltpu.VMEM((RPW, 1, L), jnp.int32),
        pltpu.VMEM((2, L, D), jnp.float32),
        pltpu.VMEM_SHARED((N, D), jnp.float32),
        pltpu.SemaphoreType.DMA,
        pltpu.SemaphoreType.DMA,
    ],
)
def _scatter(efeat_hbm, dst_hbm, mpart_hbm, idx_v, rows_v, m_sh, sem0, sem1):
    c = lax.axis_index("c")
    s = lax.axis_index("s")
    wid = _worker_id()
    sems = (sem0, sem1)
    zero = jnp.zeros((16,), jnp.float32)

    def zr(i, carry):
        for k in range(D // 16):
            rows_v[0, i, pl.ds(k * 16, 16)] = zero
        return carry

    lax.fori_loop(0, L, zr, 0)
    base_m = s * GPW * 8
    for off, n in _CHUNKS:
        pltpu.sync_copy(rows_v.at[0, pl.ds(0, n)],
                        m_sh.at[pl.ds(base_m + off, n)])

    @pl.when(s < GREM)
    def _():
        pltpu.sync_copy(rows_v.at[0, pl.ds(0, 8)],
                        m_sh.at[pl.ds((NS * GPW + s) * 8, 8)])

    plsc.subcore_barrier()

    base = wid * RPW
    pltpu.sync_copy(dst_hbm.at[pl.ds(base, RPW)], idx_v)
    pltpu.async_copy(efeat_hbm.at[pl.ds(base * L, L)], rows_v.at[0], sem0)

    def pair(g, carry):
        for b in (0, 1):
            j = 2 * g + b
            pltpu.make_async_copy(efeat_hbm.at[pl.ds((base + j) * L, L)],
                                  rows_v.at[b], sems[b]).wait()

            @pl.when(j + 1 < RPW)
            def _():
                pltpu.async_copy(
                    efeat_hbm.at[pl.ds((base + j + 1) * L, L)],
                    rows_v.at[1 - b], sems[1 - b])

            pltpu.sync_copy(rows_v.at[b], m_sh.at[idx_v.at[j, 0]], add=True)
        return carry

    lax.fori_loop(0, RPW // 2, pair, 0)

    @pl.when(wid < REM)
    def _():
        r = NW * RPW + wid
        pltpu.sync_copy(dst_hbm.at[pl.ds(r, 1)], idx_v.at[pl.ds(0, 1)])
        pltpu.sync_copy(efeat_hbm.at[pl.ds(r * L, L)], rows_v.at[0])
        pltpu.sync_copy(rows_v.at[0], m_sh.at[idx_v.at[0, 0]], add=True)

    plsc.subcore_barrier()

    for off, n in _CHUNKS:
        pltpu.sync_copy(m_sh.at[pl.ds(base_m + off, n)],
                        rows_v.at[0, pl.ds(0, n)])
        pltpu.sync_copy(rows_v.at[0, pl.ds(0, n)],
                        mpart_hbm.at[c, pl.ds(base_m + off, n)])

    @pl.when(s < GREM)
    def _():
        row = (NS * GPW + s) * 8
        pltpu.sync_copy(m_sh.at[pl.ds(row, 8)], rows_v.at[0, pl.ds(0, 8)])
        pltpu.sync_copy(rows_v.at[0, pl.ds(0, 8)],
                        mpart_hbm.at[c, pl.ds(row, 8)])


def _mw_body(mp_ref, w_ref, mw_ref):
    m = mp_ref[0] + mp_ref[1]
    mw_ref[...] = lax.dot_general(
        m, w_ref[...], (((1,), (1,)), ((), ())),
        preferred_element_type=jnp.float32)


_mw = pl.pallas_call(
    _mw_body,
    out_shape=jax.ShapeDtypeStruct((N, D), jnp.float32),
)


@functools.partial(
    pl.kernel,
    out_type=jax.ShapeDtypeStruct((E, D), jnp.float32),
    mesh=plsc.VectorSubcoreMesh(**_MESH),
    scratch_types=[
        pltpu.VMEM((RPW, 1, L), jnp.int32),
        pltpu.VMEM((2, L, D), jnp.float32),
        pltpu.VMEM_SHARED((N, D), jnp.float32),
        pltpu.SemaphoreType.DMA,
        pltpu.SemaphoreType.DMA,
    ],
)
def _gather(m_hbm, src_hbm, t_hbm, g_hbm, idx_v, rows_v, m_sh, sem0, sem1):
    del t_hbm
    s = lax.axis_index("s")
    wid = _worker_id()
    sems = (sem0, sem1)
    base_m = s * GPW * 8
    for off, n in _CHUNKS:
        pltpu.sync_copy(m_hbm.at[pl.ds(base_m + off, n)],
                        rows_v.at[0, pl.ds(0, n)])
        pltpu.sync_copy(rows_v.at[0, pl.ds(0, n)],
                        m_sh.at[pl.ds(base_m + off, n)])

    @pl.when(s < GREM)
    def _():
        row = (NS * GPW + s) * 8
        pltpu.sync_copy(m_hbm.at[pl.ds(row, 8)], rows_v.at[0, pl.ds(0, 8)])
        pltpu.sync_copy(rows_v.at[0, pl.ds(0, 8)], m_sh.at[pl.ds(row, 8)])

    plsc.subcore_barrier()
    base = wid * RPW
    pltpu.sync_copy(src_hbm.at[pl.ds(base, RPW)], idx_v)
    pltpu.async_copy(m_sh.at[idx_v.at[0, 0]], rows_v.at[0], sem0)

    def pair(g, carry):
        for b in (0, 1):
            j = 2 * g + b
            pltpu.make_async_copy(m_sh.at[idx_v.at[j, 0]],
                                  rows_v.at[b], sems[b]).wait()

            @pl.when(j + 1 < RPW)
            def _():
                pltpu.async_copy(m_sh.at[idx_v.at[j + 1, 0]],
                                 rows_v.at[1 - b], sems[1 - b])

            pltpu.sync_copy(rows_v.at[b], g_hbm.at[pl.ds((base + j) * L, L)])
        return carry

    lax.fori_loop(0, RPW // 2, pair, 0)

    @pl.when(wid < REM)
    def _():
        r = NW * RPW + wid
        pltpu.sync_copy(src_hbm.at[pl.ds(r, 1)], idx_v.at[pl.ds(0, 1)])
        pltpu.async_copy(m_sh.at[idx_v.at[0, 0]], rows_v.at[0], sem0).wait()
        pltpu.sync_copy(rows_v.at[0], g_hbm.at[pl.ds(r * L, L)])


BE = 3200


def _tkern_body(init_ref, ef_ref, w_ref, b_ref, t_ref):
    t_ref[...] = (init_ref[...] + b_ref[...] - lax.dot_general(
        ef_ref[...], w_ref[...], (((1,), (1,)), ((), ())),
        preferred_element_type=jnp.float32)).astype(jnp.bfloat16)


_tkern = pl.pallas_call(
    _tkern_body,
    grid=(E // BE,),
    in_specs=[
        pl.BlockSpec((BE, D), lambda i: (i, 0)),
        pl.BlockSpec((BE, D), lambda i: (i, 0)),
        pl.BlockSpec((D, D), lambda i: (0, 0)),
        pl.BlockSpec((1, D), lambda i: (0, 0)),
    ],
    out_specs=pl.BlockSpec((BE, D), lambda i: (i, 0)),
    out_shape=jax.ShapeDtypeStruct((E, D), jnp.bfloat16),
)


def _add_body(t_ref, g_ref, out_ref):
    out_ref[...] = t_ref[...].astype(jnp.float32) + g_ref[...]


_final_add = pl.pallas_call(
    _add_body,
    grid=(E // BE,),
    in_specs=[
        pl.BlockSpec((BE, D), lambda i: (i, 0)),
        pl.BlockSpec((BE, D), lambda i: (i, 0)),
    ],
    out_specs=pl.BlockSpec((BE, D), lambda i: (i, 0)),
    out_shape=jax.ShapeDtypeStruct((E, D), jnp.float32),
)


def kernel(efeat, initial_efeat, W, b, edge_index):
    ei = edge_index.astype(jnp.int32)
    src = ei[0].reshape(R, 1, L)
    dst = ei[1].reshape(R, 1, L)
    mpart = _scatter(efeat, dst)
    t = _tkern(initial_efeat, efeat, W, b.reshape(1, D))
    mw = _mw(mpart, W)
    g = _gather(mw, src, t)
    return _final_add(t, g)
